# baseline (device time: 156404 ns/iter reference)
import functools

import jax
import jax.numpy as jnp
from jax import lax
from jax.experimental import pallas as pl
from jax.experimental.pallas import tpu as pltpu

N_DEV = 4
_GELU_C = 0.7978845608028654


def _gelu(y):
    return 0.5 * y * (1.0 + jnp.tanh(_GELU_C * (y + 0.044715 * y * y * y)))


def kernel(x, w_mat):
    m_total, k_shard = x.shape
    k_shard2, n = w_mat.shape
    assert k_shard == k_shard2
    m_per = m_total // N_DEV

    def body(x_ref, w_ref, out_ref, sendbuf, recvbuf, send_sems, recv_sems):
        my = lax.axis_index("i")
        left = (my + N_DEV - 1) % N_DEV
        right = (my + 1) % N_DEV

        barrier_sem = pltpu.get_barrier_semaphore()
        for nbr in (left, right):
            pl.semaphore_signal(
                barrier_sem, inc=1,
                device_id=(nbr,), device_id_type=pl.DeviceIdType.MESH,
            )
        pl.semaphore_wait(barrier_sem, 2)

        def chunk_gemm(c):
            return jnp.dot(
                x_ref[pl.ds(c * m_per, m_per), :], w_ref[:, :],
                preferred_element_type=jnp.float32,
            )

        for s in range(N_DEV - 1):
            c_send = (my + N_DEV - s - 1) % N_DEV
            part = chunk_gemm(c_send)
            if s == 0:
                sendbuf[s, :, :] = part
            else:
                sendbuf[s, :, :] = recvbuf[s - 1, :, :] + part
            rdma = pltpu.make_async_remote_copy(
                src_ref=sendbuf.at[s],
                dst_ref=recvbuf.at[s],
                send_sem=send_sems.at[s],
                recv_sem=recv_sems.at[s],
                device_id=(right,),
                device_id_type=pl.DeviceIdType.MESH,
            )
            rdma.start()
            rdma.wait()

        y = recvbuf[N_DEV - 2, :, :] + chunk_gemm(my)
        out_ref[:, :] = _gelu(y)

        @functools.partial(pl.run_scoped, exit_sem=pltpu.SemaphoreType.REGULAR)
        def _(exit_sem):
            for nbr in (left, right):
                pl.semaphore_signal(
                    exit_sem, inc=1,
                    device_id=(nbr,), device_id_type=pl.DeviceIdType.MESH,
                )
            pl.semaphore_wait(exit_sem, 2)

    return pl.pallas_call(
        body,
        out_shape=jax.ShapeDtypeStruct((m_per, n), jnp.float32),
        in_specs=[
            pl.BlockSpec(memory_space=pltpu.VMEM),
            pl.BlockSpec(memory_space=pltpu.VMEM),
        ],
        out_specs=pl.BlockSpec(memory_space=pltpu.VMEM),
        scratch_shapes=[
            pltpu.VMEM((N_DEV - 1, m_per, n), jnp.float32),
            pltpu.VMEM((N_DEV - 1, m_per, n), jnp.float32),
            pltpu.SemaphoreType.DMA((N_DEV - 1,)),
            pltpu.SemaphoreType.DMA((N_DEV - 1,)),
        ],
        compiler_params=pltpu.CompilerParams(collective_id=0),
    )(x, w_mat)


# device time: 85307 ns/iter; 1.8334x vs baseline; 1.8334x over previous
import functools

import jax
import jax.numpy as jnp
from jax import lax
from jax.experimental import pallas as pl
from jax.experimental.pallas import tpu as pltpu

N_DEV = 4
_GELU_C = 0.7978845608028654


def _gelu(y):
    return 0.5 * y * (1.0 + jnp.tanh(_GELU_C * (y + 0.044715 * y * y * y)))


def kernel(x, w_mat):
    m_total, k_shard = x.shape
    k_shard2, n = w_mat.shape
    assert k_shard == k_shard2
    m_per = m_total // N_DEV
    n2 = n // 2

    def body(x_ref, w_ref, out_ref, s_r, s_l, r_r, r_l, own,
             ss_r, rs_r, ss_l, rs_l):
        my = lax.axis_index("i")
        left = (my + N_DEV - 1) % N_DEV
        right = (my + 1) % N_DEV

        barrier_sem = pltpu.get_barrier_semaphore()
        for nbr in (left, right):
            pl.semaphore_signal(
                barrier_sem, inc=1,
                device_id=(nbr,), device_id_type=pl.DeviceIdType.MESH,
            )
        pl.semaphore_wait(barrier_sem, 2)

        def half_gemm(c, col0):
            return jnp.dot(
                x_ref[pl.ds(c * m_per, m_per), :],
                w_ref[:, col0:col0 + n2],
                preferred_element_type=jnp.float32,
            )

        def rdma(s, dirn):
            if dirn == 0:
                return pltpu.make_async_remote_copy(
                    src_ref=s_r.at[s], dst_ref=r_r.at[s],
                    send_sem=ss_r.at[s], recv_sem=rs_r.at[s],
                    device_id=(right,), device_id_type=pl.DeviceIdType.MESH,
                )
            return pltpu.make_async_remote_copy(
                src_ref=s_l.at[s], dst_ref=r_l.at[s],
                send_sem=ss_l.at[s], recv_sem=rs_l.at[s],
                device_id=(left,), device_id_type=pl.DeviceIdType.MESH,
            )

        s_r[0, :, :] = half_gemm((my + 3) % N_DEV, 0)
        rd_r = rdma(0, 0)
        rd_r.start()
        s_l[0, :, :] = half_gemm((my + 1) % N_DEV, n2)
        rd_l = rdma(0, 1)
        rd_l.start()

        s_r[1, :, :] = half_gemm((my + 2) % N_DEV, 0)
        s_l[1, :, :] = half_gemm((my + 2) % N_DEV, n2)
        s_r[2, :, :] = half_gemm((my + 1) % N_DEV, 0)
        s_l[2, :, :] = half_gemm((my + 3) % N_DEV, n2)
        own[:, :] = jnp.dot(
            x_ref[pl.ds(my * m_per, m_per), :], w_ref[:, :],
            preferred_element_type=jnp.float32,
        )

        rdmas = [rd_r, rd_l]
        for s in (1, 2):
            rd_r.wait_recv()
            s_r[s, :, :] = s_r[s, :, :] + r_r[s - 1, :, :]
            rd_r = rdma(s, 0)
            rd_r.start()
            rd_l.wait_recv()
            s_l[s, :, :] = s_l[s, :, :] + r_l[s - 1, :, :]
            rd_l = rdma(s, 1)
            rd_l.start()
            rdmas += [rd_r, rd_l]

        rd_r.wait_recv()
        out_ref[:, :n2] = _gelu(own[:, :n2] + r_r[N_DEV - 2, :, :])
        rd_l.wait_recv()
        out_ref[:, n2:] = _gelu(own[:, n2:] + r_l[N_DEV - 2, :, :])

        for rd in rdmas:
            rd.wait_send()

        @functools.partial(pl.run_scoped, exit_sem=pltpu.SemaphoreType.REGULAR)
        def _(exit_sem):
            for nbr in (left, right):
                pl.semaphore_signal(
                    exit_sem, inc=1,
                    device_id=(nbr,), device_id_type=pl.DeviceIdType.MESH,
                )
            pl.semaphore_wait(exit_sem, 2)

    return pl.pallas_call(
        body,
        out_shape=jax.ShapeDtypeStruct((m_per, n), jnp.float32),
        in_specs=[
            pl.BlockSpec(memory_space=pltpu.VMEM),
            pl.BlockSpec(memory_space=pltpu.VMEM),
        ],
        out_specs=pl.BlockSpec(memory_space=pltpu.VMEM),
        scratch_shapes=[
            pltpu.VMEM((N_DEV - 1, m_per, n2), jnp.float32),
            pltpu.VMEM((N_DEV - 1, m_per, n2), jnp.float32),
            pltpu.VMEM((N_DEV - 1, m_per, n2), jnp.float32),
            pltpu.VMEM((N_DEV - 1, m_per, n2), jnp.float32),
            pltpu.VMEM((m_per, n), jnp.float32),
            pltpu.SemaphoreType.DMA((N_DEV - 1,)),
            pltpu.SemaphoreType.DMA((N_DEV - 1,)),
            pltpu.SemaphoreType.DMA((N_DEV - 1,)),
            pltpu.SemaphoreType.DMA((N_DEV - 1,)),
        ],
        compiler_params=pltpu.CompilerParams(collective_id=0),
    )(x, w_mat)


# device time: 80594 ns/iter; 1.9406x vs baseline; 1.0585x over previous
import functools

import jax
import jax.numpy as jnp
from jax import lax
from jax.experimental import pallas as pl
from jax.experimental.pallas import tpu as pltpu

N_DEV = 4
Q = 2
_GELU_C = 0.7978845608028654


def _gelu(y):
    return 0.5 * y * (1.0 + jnp.tanh(_GELU_C * (y + 0.044715 * y * y * y)))


def kernel(x, w_mat):
    m_total, k_shard = x.shape
    k_shard2, n = w_mat.shape
    assert k_shard == k_shard2
    m_per = m_total // N_DEV
    n2 = n // 2
    nq = n2 // Q

    def body(x_ref, w_ref, out_ref, s_r, s_l, r_r, r_l, own,
             ss_r, rs_r, ss_l, rs_l):
        my = lax.axis_index("i")
        left = (my + N_DEV - 1) % N_DEV
        right = (my + 1) % N_DEV

        barrier_sem = pltpu.get_barrier_semaphore()
        for nbr in (left, right):
            pl.semaphore_signal(
                barrier_sem, inc=1,
                device_id=(nbr,), device_id_type=pl.DeviceIdType.MESH,
            )
        pl.semaphore_wait(barrier_sem, 2)

        def c_right(s):
            return (my + N_DEV - s - 1) % N_DEV

        def c_left(s):
            return (my + s + 1) % N_DEV

        def col_r(q):
            return q * nq

        def col_l(q):
            return n2 + q * nq

        def sub_gemm(c, col0):
            return jnp.dot(
                x_ref[pl.ds(c * m_per, m_per), :],
                w_ref[:, col0:col0 + nq],
                preferred_element_type=jnp.float32,
            )

        def rdma(s, q, dirn):
            if dirn == 0:
                return pltpu.make_async_remote_copy(
                    src_ref=s_r.at[s, q], dst_ref=r_r.at[s, q],
                    send_sem=ss_r.at[s, q], recv_sem=rs_r.at[s, q],
                    device_id=(right,), device_id_type=pl.DeviceIdType.MESH,
                )
            return pltpu.make_async_remote_copy(
                src_ref=s_l.at[s, q], dst_ref=r_l.at[s, q],
                send_sem=ss_l.at[s, q], recv_sem=rs_l.at[s, q],
                device_id=(left,), device_id_type=pl.DeviceIdType.MESH,
            )

        rds = {}

        def start(s, q, dirn):
            rd = rdma(s, q, dirn)
            rd.start()
            rds[(s, q, dirn)] = rd

        for q in range(Q):
            s_r[0, q, :, :] = sub_gemm(c_right(0), col_r(q))
            start(0, q, 0)
            s_l[0, q, :, :] = sub_gemm(c_left(0), col_l(q))
            start(0, q, 1)

        for s in (1, 2):
            for q in range(Q):
                s_r[s, q, :, :] = sub_gemm(c_right(s), col_r(q))
                s_l[s, q, :, :] = sub_gemm(c_left(s), col_l(q))
        own[:, :] = jnp.dot(
            x_ref[pl.ds(my * m_per, m_per), :], w_ref[:, :],
            preferred_element_type=jnp.float32,
        )

        for s in (1, 2):
            for q in range(Q):
                rds[(s - 1, q, 0)].wait_recv()
                s_r[s, q, :, :] = s_r[s, q, :, :] + r_r[s - 1, q, :, :]
                start(s, q, 0)
                rds[(s - 1, q, 1)].wait_recv()
                s_l[s, q, :, :] = s_l[s, q, :, :] + r_l[s - 1, q, :, :]
                start(s, q, 1)

        last = N_DEV - 2
        for q in range(Q):
            rds[(last, q, 0)].wait_recv()
            out_ref[:, col_r(q):col_r(q) + nq] = _gelu(
                own[:, col_r(q):col_r(q) + nq] + r_r[last, q, :, :])
            rds[(last, q, 1)].wait_recv()
            out_ref[:, col_l(q):col_l(q) + nq] = _gelu(
                own[:, col_l(q):col_l(q) + nq] + r_l[last, q, :, :])

        for rd in rds.values():
            rd.wait_send()

        @functools.partial(pl.run_scoped, exit_sem=pltpu.SemaphoreType.REGULAR)
        def _(exit_sem):
            for nbr in (left, right):
                pl.semaphore_signal(
                    exit_sem, inc=1,
                    device_id=(nbr,), device_id_type=pl.DeviceIdType.MESH,
                )
            pl.semaphore_wait(exit_sem, 2)

    return pl.pallas_call(
        body,
        out_shape=jax.ShapeDtypeStruct((m_per, n), jnp.float32),
        in_specs=[
            pl.BlockSpec(memory_space=pltpu.VMEM),
            pl.BlockSpec(memory_space=pltpu.VMEM),
        ],
        out_specs=pl.BlockSpec(memory_space=pltpu.VMEM),
        scratch_shapes=[
            pltpu.VMEM((N_DEV - 1, Q, m_per, nq), jnp.float32),
            pltpu.VMEM((N_DEV - 1, Q, m_per, nq), jnp.float32),
            pltpu.VMEM((N_DEV - 1, Q, m_per, nq), jnp.float32),
            pltpu.VMEM((N_DEV - 1, Q, m_per, nq), jnp.float32),
            pltpu.VMEM((m_per, n), jnp.float32),
            pltpu.SemaphoreType.DMA((N_DEV - 1, Q)),
            pltpu.SemaphoreType.DMA((N_DEV - 1, Q)),
            pltpu.SemaphoreType.DMA((N_DEV - 1, Q)),
            pltpu.SemaphoreType.DMA((N_DEV - 1, Q)),
        ],
        compiler_params=pltpu.CompilerParams(collective_id=0),
    )(x, w_mat)


# device time: 46489 ns/iter; 3.3643x vs baseline; 1.7336x over previous
import functools

import jax
import jax.numpy as jnp
from jax import lax
from jax.experimental import pallas as pl
from jax.experimental.pallas import tpu as pltpu

N_DEV = 4
Q = 2
_GELU_C = 0.7978845608028654


def _gelu(y):
    return 0.5 * y * (1.0 + jnp.tanh(_GELU_C * (y + 0.044715 * y * y * y)))


def kernel(x, w_mat):
    m_total, k_shard = x.shape
    k_shard2, n = w_mat.shape
    assert k_shard == k_shard2
    m_per = m_total // N_DEV
    n2 = n // 2
    nq = n2 // Q

    def body(x_ref, w_ref, out_ref, s_r, s_l, r_r, r_l, pre_r, pre_l, own,
             ss_r, rs_r, ss_l, rs_l):
        my = lax.axis_index("i")
        left = (my + N_DEV - 1) % N_DEV
        right = (my + 1) % N_DEV

        barrier_sem = pltpu.get_barrier_semaphore()
        for nbr in (left, right):
            pl.semaphore_signal(
                barrier_sem, inc=1,
                device_id=(nbr,), device_id_type=pl.DeviceIdType.MESH,
            )
        pl.semaphore_wait(barrier_sem, 2)

        def c_right(s):
            return (my + N_DEV - s - 1) % N_DEV

        def c_left(s):
            return (my + s + 1) % N_DEV

        def col_r(q):
            return q * nq

        def col_l(q):
            return n2 + q * nq

        def sub_gemm(c, col0):
            return jnp.dot(
                x_ref[pl.ds(c * m_per, m_per), :],
                w_ref[:, col0:col0 + nq],
                preferred_element_type=jnp.float32,
            )

        def rdma(s, q, dirn):
            if dirn == 0:
                return pltpu.make_async_remote_copy(
                    src_ref=s_r.at[s, q], dst_ref=r_r.at[s, q],
                    send_sem=ss_r.at[s, q], recv_sem=rs_r.at[s, q],
                    device_id=(right,), device_id_type=pl.DeviceIdType.MESH,
                )
            return pltpu.make_async_remote_copy(
                src_ref=s_l.at[s, q], dst_ref=r_l.at[s, q],
                send_sem=ss_l.at[s, q], recv_sem=rs_l.at[s, q],
                device_id=(left,), device_id_type=pl.DeviceIdType.MESH,
            )

        rds = {}

        def start(s, q, dirn):
            rd = rdma(s, q, dirn)
            rd.start()
            rds[(s, q, dirn)] = rd

        for q in range(Q):
            s_r[0, q, :, :] = sub_gemm(c_right(0), col_r(q)).astype(
                jnp.bfloat16)
            start(0, q, 0)
            s_l[0, q, :, :] = sub_gemm(c_left(0), col_l(q)).astype(
                jnp.bfloat16)
            start(0, q, 1)

        for s in (1, 2):
            for q in range(Q):
                pre_r[s - 1, q, :, :] = sub_gemm(c_right(s), col_r(q))
                pre_l[s - 1, q, :, :] = sub_gemm(c_left(s), col_l(q))
        own[:, :] = jnp.dot(
            x_ref[pl.ds(my * m_per, m_per), :], w_ref[:, :],
            preferred_element_type=jnp.float32,
        )

        for s in (1, 2):
            for q in range(Q):
                rds[(s - 1, q, 0)].wait_recv()
                s_r[s, q, :, :] = (
                    pre_r[s - 1, q, :, :]
                    + r_r[s - 1, q, :, :].astype(jnp.float32)
                ).astype(jnp.bfloat16)
                start(s, q, 0)
                rds[(s - 1, q, 1)].wait_recv()
                s_l[s, q, :, :] = (
                    pre_l[s - 1, q, :, :]
                    + r_l[s - 1, q, :, :].astype(jnp.float32)
                ).astype(jnp.bfloat16)
                start(s, q, 1)

        last = N_DEV - 2
        for q in range(Q):
            rds[(last, q, 0)].wait_recv()
            out_ref[:, col_r(q):col_r(q) + nq] = _gelu(
                own[:, col_r(q):col_r(q) + nq]
                + r_r[last, q, :, :].astype(jnp.float32))
            rds[(last, q, 1)].wait_recv()
            out_ref[:, col_l(q):col_l(q) + nq] = _gelu(
                own[:, col_l(q):col_l(q) + nq]
                + r_l[last, q, :, :].astype(jnp.float32))

        for rd in rds.values():
            rd.wait_send()

        @functools.partial(pl.run_scoped, exit_sem=pltpu.SemaphoreType.REGULAR)
        def _(exit_sem):
            for nbr in (left, right):
                pl.semaphore_signal(
                    exit_sem, inc=1,
                    device_id=(nbr,), device_id_type=pl.DeviceIdType.MESH,
                )
            pl.semaphore_wait(exit_sem, 2)

    return pl.pallas_call(
        body,
        out_shape=jax.ShapeDtypeStruct((m_per, n), jnp.float32),
        in_specs=[
            pl.BlockSpec(memory_space=pltpu.VMEM),
            pl.BlockSpec(memory_space=pltpu.VMEM),
        ],
        out_specs=pl.BlockSpec(memory_space=pltpu.VMEM),
        scratch_shapes=[
            pltpu.VMEM((N_DEV - 1, Q, m_per, nq), jnp.bfloat16),
            pltpu.VMEM((N_DEV - 1, Q, m_per, nq), jnp.bfloat16),
            pltpu.VMEM((N_DEV - 1, Q, m_per, nq), jnp.bfloat16),
            pltpu.VMEM((N_DEV - 1, Q, m_per, nq), jnp.bfloat16),
            pltpu.VMEM((N_DEV - 2, Q, m_per, nq), jnp.float32),
            pltpu.VMEM((N_DEV - 2, Q, m_per, nq), jnp.float32),
            pltpu.VMEM((m_per, n), jnp.float32),
            pltpu.SemaphoreType.DMA((N_DEV - 1, Q)),
            pltpu.SemaphoreType.DMA((N_DEV - 1, Q)),
            pltpu.SemaphoreType.DMA((N_DEV - 1, Q)),
            pltpu.SemaphoreType.DMA((N_DEV - 1, Q)),
        ],
        compiler_params=pltpu.CompilerParams(collective_id=0),
    )(x, w_mat)
